# baseline (speedup 1.0000x reference)
import functools
import math

import jax
import jax.numpy as jnp
from jax import lax
from jax.experimental import pallas as pl
from jax.experimental.pallas import tpu as pltpu
from jax.experimental.pallas import tpu_sc as plsc

_B = 16
_T = 512
_ZD = 256
_CD = 2048
_NS = 12
_NNEG = 10
_LEN = _T - _NS
_TP = 512
_NCLS = 1 + _NNEG

_CHUNK = 128
_NCH = _TP // _CHUNK
_NW = 32
_PER_W = (_NS * _B * _NCH) // _NW


def _neg_indices():
    idx_key = jax.random.key(42)
    per_k = []
    for k in range(1, _NS + 1):
        kk = jax.random.fold_in(idx_key, k)
        seq = jax.random.randint(kk, (_B, _NNEG, _LEN), 1, _LEN)
        seq = jnp.remainder(seq + jnp.arange(_LEN), _LEN)
        cols = seq + k
        diag = jnp.broadcast_to((jnp.arange(_LEN) + k)[None, None, :],
                                (_B, 1, _LEN))
        full = jnp.concatenate([diag, cols], axis=1)
        full = jnp.pad(full, ((0, 0), (0, 0), (0, _TP - _LEN)))
        per_k.append(full)
    return jnp.stack(per_k).astype(jnp.int32)


_DN = (((1,), (1,)), ((), ()))


def _split_bf16(x):
    hi = x.astype(jnp.bfloat16)
    lo = (x - hi.astype(jnp.float32)).astype(jnp.bfloat16)
    return hi, lo


def _dot3(a, b):
    a_hi, a_lo = _split_bf16(a)
    b_hi, b_lo = _split_bf16(b)
    f32 = jnp.float32
    out = lax.dot_general(a_hi, b_hi, _DN, preferred_element_type=f32)
    out += lax.dot_general(a_hi, b_lo, _DN, preferred_element_type=f32)
    out += lax.dot_general(a_lo, b_hi, _DN, preferred_element_type=f32)
    return out


def _mm_body(ch_ref, cl_ref, zh_ref, zl_ref, wh_ref, wl_ref, b_ref, g_ref):
    k = pl.program_id(1)
    f32 = jnp.float32
    wc = lax.dot_general(ch_ref[0], wh_ref[k], _DN, preferred_element_type=f32)
    wc += lax.dot_general(ch_ref[0], wl_ref[k], _DN, preferred_element_type=f32)
    wc += lax.dot_general(cl_ref[0], wh_ref[k], _DN, preferred_element_type=f32)
    wc += b_ref[k]
    wc_hi, wc_lo = _split_bf16(wc)
    g = lax.dot_general(wc_hi, zh_ref[0], _DN, preferred_element_type=f32)
    g += lax.dot_general(wc_hi, zl_ref[0], _DN, preferred_element_type=f32)
    g += lax.dot_general(wc_lo, zh_ref[0], _DN, preferred_element_type=f32)
    g_ref[0, 0] = g * (1.0 / math.sqrt(_ZD))


def _scores(z, c, W, b):
    c_hi = c.astype(jnp.bfloat16)
    c_lo = (c - c_hi.astype(jnp.float32)).astype(jnp.bfloat16)
    z_hi = z.astype(jnp.bfloat16)
    z_lo = (z - z_hi.astype(jnp.float32)).astype(jnp.bfloat16)
    w_hi = W.astype(jnp.bfloat16)
    w_lo = (W - w_hi.astype(jnp.float32)).astype(jnp.bfloat16)
    bspec = lambda bb, kk: (bb, 0, 0)
    wspec = lambda bb, kk: (0, 0, 0)
    return pl.pallas_call(
        _mm_body,
        grid=(_B, _NS),
        in_specs=[
            pl.BlockSpec((1, _T, _CD), bspec),
            pl.BlockSpec((1, _T, _CD), bspec),
            pl.BlockSpec((1, _T, _ZD), bspec),
            pl.BlockSpec((1, _T, _ZD), bspec),
            pl.BlockSpec((_NS, _ZD, _CD), wspec),
            pl.BlockSpec((_NS, _ZD, _CD), wspec),
            pl.BlockSpec((_NS, 1, _ZD), wspec),
        ],
        out_specs=pl.BlockSpec((1, 1, _TP, _TP), lambda bb, kk: (kk, bb, 0, 0)),
        out_shape=jax.ShapeDtypeStruct((_NS, _B, _TP, _TP), jnp.float32),
    )(c_hi, c_lo, z_hi, z_lo, w_hi, w_lo, b.reshape(_NS, 1, _ZD))


def _sc_body(g_hbm, idx_hbm, out_hbm, gbuf, ibuf, obuf):
    wid = lax.axis_index("s") * 2 + lax.axis_index("c")

    def chunk(i, carry):
        cid = wid * _PER_W + i
        kb = lax.div(cid, _NCH)
        tcoff = lax.rem(cid, _NCH) * _CHUNK
        pltpu.sync_copy(g_hbm.at[pl.ds(cid * (_CHUNK * _TP), _CHUNK * _TP)],
                        gbuf)
        pltpu.sync_copy(idx_hbm.at[pl.ds(kb * (_NCLS * _TP), _NCLS * _TP)],
                        ibuf)

        def group(gi, c2):
            base = gi * 16
            rowoff = (lax.iota(jnp.int32, 16) + base) * _TP
            vals = [plsc.load_gather(
                        gbuf,
                        [rowoff + ibuf[pl.ds(j * _TP + tcoff + base, 16)]])
                    for j in range(_NCLS)]
            m = vals[0]
            for j in range(1, _NCLS):
                m = jnp.maximum(m, vals[j])
            s = jnp.exp(vals[0] - m)
            for j in range(1, _NCLS):
                s = s + jnp.exp(vals[j] - m)
            obuf[pl.ds(base, 16)] = m
            obuf[pl.ds(_CHUNK + base, 16)] = s
            obuf[pl.ds(2 * _CHUNK + base, 16)] = vals[0]
            obuf[pl.ds(3 * _CHUNK + base, 16)] = jnp.zeros((16,), jnp.float32)
            return c2

        lax.fori_loop(0, _CHUNK // 16, group, 0)
        pltpu.sync_copy(obuf, out_hbm.at[pl.ds(cid * (4 * _CHUNK), 4 * _CHUNK)])
        return carry

    lax.fori_loop(0, _PER_W, chunk, 0)


def _sc_gather(g, idx):
    fn = functools.partial(
        pl.kernel,
        out_type=jax.ShapeDtypeStruct((_NS * _B * _NCH * 4 * _CHUNK,),
                                      jnp.float32),
        mesh=plsc.VectorSubcoreMesh(core_axis_name="c", subcore_axis_name="s"),
        compiler_params=pltpu.CompilerParams(needs_layout_passes=False),
        scratch_types=[
            pltpu.VMEM((_CHUNK * _TP,), jnp.float32),
            pltpu.VMEM((_NCLS * _TP,), jnp.int32),
            pltpu.VMEM((4 * _CHUNK,), jnp.float32),
        ],
    )(_sc_body)
    return fn(g.reshape(_NS * _B * _NCH * _CHUNK * _TP),
              idx.reshape(_NS * _B * _NCLS * _TP))


def _fin_body(m_ref, s_ref, f0_ref, loss_ref, acc_ref):
    m = m_ref[...]
    s = s_ref[...]
    f0 = f0_ref[...]
    col = lax.broadcasted_iota(jnp.int32, (_NS, _B * _TP), 1)
    valid = (col % _TP) < _LEN
    ce = jnp.where(valid, m + jnp.log(s) - f0, 0.0)
    loss_ref[...] = jnp.reshape(jnp.sum(ce) / (_NS * _B * _LEN), (1, 1))
    ind = jnp.where(valid & (f0 >= m), 1.0, 0.0)
    acc_ref[...] = (jnp.sum(ind, axis=1) / (_B * _LEN))[:, None]


def _finalize(m2, s2, f02):
    return pl.pallas_call(
        _fin_body,
        out_shape=[jax.ShapeDtypeStruct((1, 1), jnp.float32),
                   jax.ShapeDtypeStruct((_NS, 1), jnp.float32)],
    )(m2, s2, f02)


def kernel(z, c, W, b):
    idx = _neg_indices()
    g = _scores(z, c, W, b)
    out = _sc_gather(g, idx).reshape(_NS, _B, _NCH, 4, _CHUNK)
    m2 = out[:, :, :, 0, :].reshape(_NS, _B * _TP)
    s2 = out[:, :, :, 1, :].reshape(_NS, _B * _TP)
    f02 = out[:, :, :, 2, :].reshape(_NS, _B * _TP)
    loss, accs = _finalize(m2, s2, f02)
    return loss[0, 0], accs[:, 0]


# speedup vs baseline: 1.0195x; 1.0195x over previous
import functools
import math

import jax
import jax.numpy as jnp
from jax import lax
from jax.experimental import pallas as pl
from jax.experimental.pallas import tpu as pltpu
from jax.experimental.pallas import tpu_sc as plsc

_B = 16
_T = 512
_ZD = 256
_CD = 2048
_NS = 12
_NNEG = 10
_LEN = _T - _NS
_TP = 512
_NCLS = 1 + _NNEG

_CHUNK = 128
_NCH = _TP // _CHUNK
_NW = 32
_PER_W = (_NS * _B * _NCH) // _NW


def _neg_indices():
    idx_key = jax.random.key(42)
    per_k = []
    for k in range(1, _NS + 1):
        kk = jax.random.fold_in(idx_key, k)
        seq = jax.random.randint(kk, (_B, _NNEG, _LEN), 1, _LEN)
        seq = jnp.remainder(seq + jnp.arange(_LEN), _LEN)
        cols = seq + k
        diag = jnp.broadcast_to((jnp.arange(_LEN) + k)[None, None, :],
                                (_B, 1, _LEN))
        full = jnp.concatenate([diag, cols], axis=1)
        full = jnp.pad(full, ((0, 0), (0, 0), (0, _TP - _LEN)))
        per_k.append(full)
    return jnp.stack(per_k).astype(jnp.int32)


_DN = (((1,), (1,)), ((), ()))


def _split_bf16(x):
    hi = x.astype(jnp.bfloat16)
    lo = (x - hi.astype(jnp.float32)).astype(jnp.bfloat16)
    return hi, lo


def _dot3(a, b):
    a_hi, a_lo = _split_bf16(a)
    b_hi, b_lo = _split_bf16(b)
    f32 = jnp.float32
    out = lax.dot_general(a_hi, b_hi, _DN, preferred_element_type=f32)
    out += lax.dot_general(a_hi, b_lo, _DN, preferred_element_type=f32)
    out += lax.dot_general(a_lo, b_hi, _DN, preferred_element_type=f32)
    return out


def _mm_body(c_ref, z_ref, w_ref, b_ref, g_ref):
    k = pl.program_id(1)
    cb = c_ref[0]
    zb = z_ref[0]
    wk = w_ref[k]
    wc = _dot3(cb, wk) + b_ref[k]
    g = _dot3(wc, zb)
    g_ref[0, 0] = g * (1.0 / math.sqrt(_ZD))


def _scores(z, c, W, b, ns):
    return pl.pallas_call(
        _mm_body,
        grid=(_B, ns),
        in_specs=[
            pl.BlockSpec((1, _T, _CD), lambda bb, kk: (bb, 0, 0)),
            pl.BlockSpec((1, _T, _ZD), lambda bb, kk: (bb, 0, 0)),
            pl.BlockSpec((ns, _ZD, _CD), lambda bb, kk: (0, 0, 0)),
            pl.BlockSpec((ns, 1, _ZD), lambda bb, kk: (0, 0, 0)),
        ],
        out_specs=pl.BlockSpec((1, 1, _TP, _TP), lambda bb, kk: (kk, bb, 0, 0)),
        out_shape=jax.ShapeDtypeStruct((ns, _B, _TP, _TP), jnp.float32),
    )(c, z, W, b.reshape(ns, 1, _ZD))


def _sc_body(per_w, g_hbm, idx_hbm, out_hbm, gbuf, ibuf, obuf):
    wid = lax.axis_index("s") * 2 + lax.axis_index("c")

    def chunk(i, carry):
        cid = wid * per_w + i
        kb = lax.div(cid, _NCH)
        tcoff = lax.rem(cid, _NCH) * _CHUNK
        pltpu.sync_copy(g_hbm.at[pl.ds(cid * (_CHUNK * _TP), _CHUNK * _TP)],
                        gbuf)
        pltpu.sync_copy(idx_hbm.at[pl.ds(kb * (_NCLS * _TP), _NCLS * _TP)],
                        ibuf)

        def group(gi, c2):
            base = gi * 16
            rowoff = (lax.iota(jnp.int32, 16) + base) * _TP
            vals = [plsc.load_gather(
                        gbuf,
                        [rowoff + ibuf[pl.ds(j * _TP + tcoff + base, 16)]])
                    for j in range(_NCLS)]
            m = vals[0]
            for j in range(1, _NCLS):
                m = jnp.maximum(m, vals[j])
            s = jnp.exp(vals[0] - m)
            for j in range(1, _NCLS):
                s = s + jnp.exp(vals[j] - m)
            obuf[pl.ds(base, 16)] = m
            obuf[pl.ds(_CHUNK + base, 16)] = s
            obuf[pl.ds(2 * _CHUNK + base, 16)] = vals[0]
            obuf[pl.ds(3 * _CHUNK + base, 16)] = jnp.zeros((16,), jnp.float32)
            return c2

        lax.fori_loop(0, _CHUNK // 16, group, 0)
        pltpu.sync_copy(obuf, out_hbm.at[pl.ds(cid * (4 * _CHUNK), 4 * _CHUNK)])
        return carry

    lax.fori_loop(0, per_w, chunk, 0)


def _sc_gather(g, idx, ns):
    per_w = (ns * _B * _NCH) // _NW
    fn = functools.partial(
        pl.kernel,
        out_type=jax.ShapeDtypeStruct((ns * _B * _NCH * 4 * _CHUNK,),
                                      jnp.float32),
        mesh=plsc.VectorSubcoreMesh(core_axis_name="c", subcore_axis_name="s"),
        compiler_params=pltpu.CompilerParams(needs_layout_passes=False),
        scratch_types=[
            pltpu.VMEM((_CHUNK * _TP,), jnp.float32),
            pltpu.VMEM((_NCLS * _TP,), jnp.int32),
            pltpu.VMEM((4 * _CHUNK,), jnp.float32),
        ],
    )(functools.partial(_sc_body, per_w))
    return fn(g.reshape(ns * _B * _NCH * _CHUNK * _TP),
              idx.reshape(ns * _B * _NCLS * _TP))


def _fin_body(m_ref, s_ref, f0_ref, loss_ref, acc_ref):
    m = m_ref[...]
    s = s_ref[...]
    f0 = f0_ref[...]
    col = lax.broadcasted_iota(jnp.int32, (_NS, _B * _TP), 1)
    valid = (col % _TP) < _LEN
    ce = jnp.where(valid, m + jnp.log(s) - f0, 0.0)
    loss_ref[...] = jnp.reshape(jnp.sum(ce) / (_NS * _B * _LEN), (1, 1))
    ind = jnp.where(valid & (f0 >= m), 1.0, 0.0)
    acc_ref[...] = (jnp.sum(ind, axis=1) / (_B * _LEN))[:, None]


def _finalize(m2, s2, f02):
    return pl.pallas_call(
        _fin_body,
        out_shape=[jax.ShapeDtypeStruct((1, 1), jnp.float32),
                   jax.ShapeDtypeStruct((_NS, 1), jnp.float32)],
    )(m2, s2, f02)


def kernel(z, c, W, b):
    idx = _neg_indices()
    h = _NS // 2
    g1 = _scores(z, c, W[:h], b[:h], h)
    g2 = _scores(z, c, W[h:], b[h:], h)
    o1 = _sc_gather(g1, idx[:h], h)
    o2 = _sc_gather(g2, idx[h:], h)
    out = jnp.concatenate([o1, o2]).reshape(_NS, _B, _NCH, 4, _CHUNK)
    m2 = out[:, :, :, 0, :].reshape(_NS, _B * _TP)
    s2 = out[:, :, :, 1, :].reshape(_NS, _B * _TP)
    f02 = out[:, :, :, 2, :].reshape(_NS, _B * _TP)
    loss, accs = _finalize(m2, s2, f02)
    return loss[0, 0], accs[:, 0]


# speedup vs baseline: 1.0705x; 1.0500x over previous
import functools
import math

import jax
import jax.numpy as jnp
from jax import lax
from jax.experimental import pallas as pl
from jax.experimental.pallas import tpu as pltpu
from jax.experimental.pallas import tpu_sc as plsc

_B = 16
_T = 512
_ZD = 256
_CD = 2048
_NS = 12
_NNEG = 10
_LEN = _T - _NS
_TP = 512
_NCLS = 1 + _NNEG

_CHUNK = 128
_NCH = _TP // _CHUNK
_NW = 32
_PER_W = (_NS * _B * _NCH) // _NW


def _neg_indices():
    idx_key = jax.random.key(42)
    per_k = []
    for k in range(1, _NS + 1):
        kk = jax.random.fold_in(idx_key, k)
        seq = jax.random.randint(kk, (_B, _NNEG, _LEN), 1, _LEN)
        seq = jnp.remainder(seq + jnp.arange(_LEN), _LEN)
        cols = seq + k
        diag = jnp.broadcast_to((jnp.arange(_LEN) + k)[None, None, :],
                                (_B, 1, _LEN))
        full = jnp.concatenate([diag, cols], axis=1)
        full = jnp.pad(full, ((0, 0), (0, 0), (0, _TP - _LEN)))
        per_k.append(full)
    return jnp.stack(per_k).astype(jnp.int32)


_DN = (((1,), (1,)), ((), ()))


def _split_bf16(x):
    hi = x.astype(jnp.bfloat16)
    lo = (x - hi.astype(jnp.float32)).astype(jnp.bfloat16)
    return hi, lo


def _dot3(a, b):
    a_hi, a_lo = _split_bf16(a)
    b_hi, b_lo = _split_bf16(b)
    f32 = jnp.float32
    out = lax.dot_general(a_hi, b_hi, _DN, preferred_element_type=f32)
    out += lax.dot_general(a_hi, b_lo, _DN, preferred_element_type=f32)
    out += lax.dot_general(a_lo, b_hi, _DN, preferred_element_type=f32)
    return out


def _mm_body(c_ref, z_ref, w_ref, b_ref, g_ref):
    k = pl.program_id(1)
    cb = c_ref[0]
    zb = z_ref[0]
    wk = w_ref[k]
    wc = _dot3(cb, wk) + b_ref[k]
    g = _dot3(wc, zb)
    g_ref[0, 0] = g * (1.0 / math.sqrt(_ZD))


def _scores(z, c, W, b, ns):
    return pl.pallas_call(
        _mm_body,
        grid=(_B, ns),
        in_specs=[
            pl.BlockSpec((1, _T, _CD), lambda bb, kk: (bb, 0, 0)),
            pl.BlockSpec((1, _T, _ZD), lambda bb, kk: (bb, 0, 0)),
            pl.BlockSpec((ns, _ZD, _CD), lambda bb, kk: (0, 0, 0)),
            pl.BlockSpec((ns, 1, _ZD), lambda bb, kk: (0, 0, 0)),
        ],
        out_specs=pl.BlockSpec((1, 1, _TP, _TP), lambda bb, kk: (kk, bb, 0, 0)),
        out_shape=jax.ShapeDtypeStruct((ns, _B, _TP, _TP), jnp.float32),
    )(c, z, W, b.reshape(ns, 1, _ZD))


def _sc_body(per_w, g_hbm, idx_hbm, out_hbm, gbuf, ibuf, obuf):
    wid = lax.axis_index("s") * 2 + lax.axis_index("c")

    def chunk(i, carry):
        cid = wid * per_w + i
        kb = lax.div(cid, _NCH)
        tc = lax.rem(cid, _NCH)
        bb = lax.rem(kb, _B)
        kk = lax.div(kb, _B)
        tcoff = tc * _CHUNK
        pltpu.sync_copy(g_hbm.at[kk, bb, pl.ds(tcoff, _CHUNK)], gbuf)
        pltpu.sync_copy(idx_hbm.at[pl.ds(kb * (_NCLS * _TP), _NCLS * _TP)],
                        ibuf)

        def group(gi, c2):
            base = gi * 16
            row = lax.iota(jnp.int32, 16) + base
            vals = [plsc.load_gather(
                        gbuf,
                        [row, ibuf[pl.ds(j * _TP + tcoff + base, 16)]])
                    for j in range(_NCLS)]
            m = vals[0]
            for j in range(1, _NCLS):
                m = jnp.maximum(m, vals[j])
            s = jnp.exp(vals[0] - m)
            for j in range(1, _NCLS):
                s = s + jnp.exp(vals[j] - m)
            obuf[pl.ds(base, 16)] = m
            obuf[pl.ds(_CHUNK + base, 16)] = s
            obuf[pl.ds(2 * _CHUNK + base, 16)] = vals[0]
            obuf[pl.ds(3 * _CHUNK + base, 16)] = jnp.zeros((16,), jnp.float32)
            return c2

        lax.fori_loop(0, _CHUNK // 16, group, 0)
        pltpu.sync_copy(obuf, out_hbm.at[pl.ds(cid * (4 * _CHUNK), 4 * _CHUNK)])
        return carry

    lax.fori_loop(0, per_w, chunk, 0)


def _sc_gather(g, idx, ns):
    per_w = (ns * _B * _NCH) // _NW
    fn = functools.partial(
        pl.kernel,
        out_type=jax.ShapeDtypeStruct((ns * _B * _NCH * 4 * _CHUNK,),
                                      jnp.float32),
        mesh=plsc.VectorSubcoreMesh(core_axis_name="c", subcore_axis_name="s"),
        compiler_params=pltpu.CompilerParams(needs_layout_passes=False),
        scratch_types=[
            pltpu.VMEM((_CHUNK, _TP), jnp.float32),
            pltpu.VMEM((_NCLS * _TP,), jnp.int32),
            pltpu.VMEM((4 * _CHUNK,), jnp.float32),
        ],
    )(functools.partial(_sc_body, per_w))
    return fn(g, idx.reshape(ns * _B * _NCLS * _TP))


def _fin_body(m_ref, s_ref, f0_ref, loss_ref, acc_ref):
    m = m_ref[...]
    s = s_ref[...]
    f0 = f0_ref[...]
    col = lax.broadcasted_iota(jnp.int32, (_NS, _B * _TP), 1)
    valid = (col % _TP) < _LEN
    ce = jnp.where(valid, m + jnp.log(s) - f0, 0.0)
    loss_ref[...] = jnp.reshape(jnp.sum(ce) / (_NS * _B * _LEN), (1, 1))
    ind = jnp.where(valid & (f0 >= m), 1.0, 0.0)
    acc_ref[...] = (jnp.sum(ind, axis=1) / (_B * _LEN))[:, None]


def _finalize(m2, s2, f02):
    return pl.pallas_call(
        _fin_body,
        out_shape=[jax.ShapeDtypeStruct((1, 1), jnp.float32),
                   jax.ShapeDtypeStruct((_NS, 1), jnp.float32)],
    )(m2, s2, f02)


def kernel(z, c, W, b):
    idx = _neg_indices()
    g = _scores(z, c, W, b, _NS)
    out = _sc_gather(g, idx, _NS).reshape(_NS, _B, _NCH, 4, _CHUNK)
    m2 = out[:, :, :, 0, :].reshape(_NS, _B * _TP)
    s2 = out[:, :, :, 1, :].reshape(_NS, _B * _TP)
    f02 = out[:, :, :, 2, :].reshape(_NS, _B * _TP)
    loss, accs = _finalize(m2, s2, f02)
    return loss[0, 0], accs[:, 0]


# speedup vs baseline: 1.0916x; 1.0197x over previous
import functools
import math

import jax
import jax.numpy as jnp
from jax import lax
from jax.experimental import pallas as pl
from jax.experimental.pallas import tpu as pltpu
from jax.experimental.pallas import tpu_sc as plsc

_B = 16
_T = 512
_ZD = 256
_CD = 2048
_NS = 12
_NNEG = 10
_LEN = _T - _NS
_TP = 512
_NCLS = 1 + _NNEG

_CHUNK = 128
_NCH = _TP // _CHUNK
_NW = 32
_PER_W = (_NS * _B * _NCH) // _NW


def _neg_indices():
    idx_key = jax.random.key(42)
    per_k = []
    for k in range(1, _NS + 1):
        kk = jax.random.fold_in(idx_key, k)
        seq = jax.random.randint(kk, (_B, _NNEG, _LEN), 1, _LEN)
        seq = jnp.remainder(seq + jnp.arange(_LEN), _LEN)
        cols = seq + k
        diag = jnp.broadcast_to((jnp.arange(_LEN) + k)[None, None, :],
                                (_B, 1, _LEN))
        full = jnp.concatenate([diag, cols], axis=1)
        full = jnp.pad(full, ((0, 0), (0, 0), (0, _TP - _LEN)))
        per_k.append(full)
    return jnp.stack(per_k).astype(jnp.int32)


_DN = (((1,), (0,)), ((), ()))


def _split_bf16(x):
    hi = x.astype(jnp.bfloat16)
    lo = (x - hi.astype(jnp.float32)).astype(jnp.bfloat16)
    return hi, lo


def _dot3(a, b):
    a_hi, a_lo = _split_bf16(a)
    b_hi, b_lo = _split_bf16(b)
    f32 = jnp.float32
    out = lax.dot_general(a_hi, b_hi, _DN, preferred_element_type=f32)
    out += lax.dot_general(a_hi, b_lo, _DN, preferred_element_type=f32)
    out += lax.dot_general(a_lo, b_hi, _DN, preferred_element_type=f32)
    return out


def _mm_body(c_ref, z_ref, w_ref, b_ref, g_ref):
    k = pl.program_id(1)
    cb = c_ref[0]
    zbt = z_ref[0]
    wkt = w_ref[k]
    wc = _dot3(cb, wkt) + b_ref[k]
    g = _dot3(wc, zbt)
    g_ref[0, 0] = g * (1.0 / math.sqrt(_ZD))


def _scores(z, c, W, b, ns):
    w_t = jnp.swapaxes(W, 1, 2)
    z_t = jnp.swapaxes(z, 1, 2)
    return pl.pallas_call(
        _mm_body,
        grid=(_B, ns),
        in_specs=[
            pl.BlockSpec((1, _T, _CD), lambda bb, kk: (bb, 0, 0)),
            pl.BlockSpec((1, _ZD, _TP), lambda bb, kk: (bb, 0, 0)),
            pl.BlockSpec((ns, _CD, _ZD), lambda bb, kk: (0, 0, 0)),
            pl.BlockSpec((ns, 1, _ZD), lambda bb, kk: (0, 0, 0)),
        ],
        out_specs=pl.BlockSpec((1, 1, _TP, _TP), lambda bb, kk: (kk, bb, 0, 0)),
        out_shape=jax.ShapeDtypeStruct((ns, _B, _TP, _TP), jnp.float32),
    )(c, z_t, w_t, b.reshape(ns, 1, _ZD))


def _sc_body(per_w, g_hbm, idx_hbm, out_hbm, gbuf, ibuf, obuf):
    wid = lax.axis_index("s") * 2 + lax.axis_index("c")

    def chunk(i, carry):
        cid = wid * per_w + i
        kb = lax.div(cid, _NCH)
        tc = lax.rem(cid, _NCH)
        bb = lax.rem(kb, _B)
        kk = lax.div(kb, _B)
        tcoff = tc * _CHUNK
        pltpu.sync_copy(g_hbm.at[kk, bb, pl.ds(tcoff, _CHUNK)], gbuf)
        pltpu.sync_copy(idx_hbm.at[pl.ds(kb * (_NCLS * _TP), _NCLS * _TP)],
                        ibuf)

        def group(gi, c2):
            base = gi * 16
            row = lax.iota(jnp.int32, 16) + base
            vals = [plsc.load_gather(
                        gbuf,
                        [row, ibuf[pl.ds(j * _TP + tcoff + base, 16)]])
                    for j in range(_NCLS)]
            m = vals[0]
            for j in range(1, _NCLS):
                m = jnp.maximum(m, vals[j])
            s = jnp.exp(vals[0] - m)
            for j in range(1, _NCLS):
                s = s + jnp.exp(vals[j] - m)
            obuf[pl.ds(base, 16)] = m
            obuf[pl.ds(_CHUNK + base, 16)] = s
            obuf[pl.ds(2 * _CHUNK + base, 16)] = vals[0]
            obuf[pl.ds(3 * _CHUNK + base, 16)] = jnp.zeros((16,), jnp.float32)
            return c2

        lax.fori_loop(0, _CHUNK // 16, group, 0)
        pltpu.sync_copy(obuf, out_hbm.at[pl.ds(cid * (4 * _CHUNK), 4 * _CHUNK)])
        return carry

    lax.fori_loop(0, per_w, chunk, 0)


def _sc_gather(g, idx, ns):
    per_w = (ns * _B * _NCH) // _NW
    fn = functools.partial(
        pl.kernel,
        out_type=jax.ShapeDtypeStruct((ns * _B * _NCH * 4 * _CHUNK,),
                                      jnp.float32),
        mesh=plsc.VectorSubcoreMesh(core_axis_name="c", subcore_axis_name="s"),
        compiler_params=pltpu.CompilerParams(needs_layout_passes=False),
        scratch_types=[
            pltpu.VMEM((_CHUNK, _TP), jnp.float32),
            pltpu.VMEM((_NCLS * _TP,), jnp.int32),
            pltpu.VMEM((4 * _CHUNK,), jnp.float32),
        ],
    )(functools.partial(_sc_body, per_w))
    return fn(g, idx.reshape(ns * _B * _NCLS * _TP))


def _fin_body(m_ref, s_ref, f0_ref, loss_ref, acc_ref):
    m = m_ref[...]
    s = s_ref[...]
    f0 = f0_ref[...]
    col = lax.broadcasted_iota(jnp.int32, (_NS, _B * _TP), 1)
    valid = (col % _TP) < _LEN
    ce = jnp.where(valid, m + jnp.log(s) - f0, 0.0)
    loss_ref[...] = jnp.reshape(jnp.sum(ce) / (_NS * _B * _LEN), (1, 1))
    ind = jnp.where(valid & (f0 >= m), 1.0, 0.0)
    acc_ref[...] = (jnp.sum(ind, axis=1) / (_B * _LEN))[:, None]


def _finalize(m2, s2, f02):
    return pl.pallas_call(
        _fin_body,
        out_shape=[jax.ShapeDtypeStruct((1, 1), jnp.float32),
                   jax.ShapeDtypeStruct((_NS, 1), jnp.float32)],
    )(m2, s2, f02)


def kernel(z, c, W, b):
    idx = _neg_indices()
    g = _scores(z, c, W, b, _NS)
    out = _sc_gather(g, idx, _NS).reshape(_NS, _B, _NCH, 4, _CHUNK)
    m2 = out[:, :, :, 0, :].reshape(_NS, _B * _TP)
    s2 = out[:, :, :, 1, :].reshape(_NS, _B * _TP)
    f02 = out[:, :, :, 2, :].reshape(_NS, _B * _TP)
    loss, accs = _finalize(m2, s2, f02)
    return loss[0, 0], accs[:, 0]
